# baseline (speedup 1.0000x reference)
import functools

import jax
import jax.numpy as jnp
from jax import lax
from jax.experimental import pallas as pl
from jax.experimental.pallas import tpu as pltpu

IC = 128
NGROUP = 32
CPG = IC // NGROUP
EPSV = 1e-5
NSEG = 16


def _dot_t(a, b):
    return lax.dot_general(a, b, (((0,), (0,)), ((), ())),
                           preferred_element_type=jnp.float32)


def _onehot(bid_col, rows):
    seg = lax.broadcasted_iota(jnp.int32, (rows, NSEG), 1)
    return (bid_col == seg).astype(jnp.float32)


def _seg_row_mask(seg_id):
    rowi = lax.broadcasted_iota(jnp.int32, (NSEG, IC), 0)
    return (rowi == seg_id).astype(jnp.float32)


def _body(nblocks, rows, bfirst_ref, blast_ref, x_ref, bid_ref, w_ref,
          b_ref, o_ref, xs, s1, s2, cnt):
    p = pl.program_id(0)
    j = pl.program_id(1)
    lo = bfirst_ref[j]
    uniform = lo == blast_ref[j]

    @pl.when((p == 0) & (j == 0))
    def _():
        s1[...] = jnp.zeros_like(s1)
        s2[...] = jnp.zeros_like(s2)
        cnt[...] = jnp.zeros_like(cnt)

    @pl.when(p == 0)
    def _():
        x = x_ref[...]
        xs[pl.ds(j * rows, rows), :] = x

        @pl.when(uniform)
        def _():
            m = _seg_row_mask(lo)
            s1[...] += m * jnp.sum(x, axis=0, keepdims=True)
            s2[...] += m * jnp.sum(x * x, axis=0, keepdims=True)
            cnt[...] += m * jnp.float32(rows)

        @pl.when(jnp.logical_not(uniform))
        def _():
            oh = _onehot(bid_ref[...], rows)
            s1[...] += _dot_t(oh, x)
            s2[...] += _dot_t(oh, x * x)
            cnt[...] += _dot_t(oh, jnp.ones_like(x))

        @pl.when(j == nblocks - 1)
        def _():
            ic = 1.0 / (cnt[...] * CPG + EPSV)
            ci = lax.broadcasted_iota(jnp.int32, (IC, IC), 0) // CPG
            cj = lax.broadcasted_iota(jnp.int32, (IC, IC), 1) // CPG
            ggt = (ci == cj).astype(jnp.float32)
            a1 = lax.dot_general(s1[...], ggt, (((1,), (0,)), ((), ())),
                                 preferred_element_type=jnp.float32)
            a2 = lax.dot_general(s2[...], ggt, (((1,), (0,)), ((), ())),
                                 preferred_element_type=jnp.float32)
            mg = a1 * ic
            var = ic * (a2 - 2.0 * mg * a1 + cnt[...] * CPG * mg * mg)
            istd = lax.rsqrt(var + EPSV)
            w = w_ref[...]
            scale = istd * w
            shift = b_ref[...] - mg * scale
            s1[...] = scale
            s2[...] = shift

    @pl.when(p == 1)
    def _():
        x = xs[pl.ds(j * rows, rows), :]

        @pl.when(uniform)
        def _():
            m = _seg_row_mask(lo)
            rs = jnp.sum(m * s1[...], axis=0, keepdims=True)
            rh = jnp.sum(m * s2[...], axis=0, keepdims=True)
            o_ref[...] = x * rs + rh

        @pl.when(jnp.logical_not(uniform))
        def _():
            oh = _onehot(bid_ref[...], rows)
            rs = lax.dot_general(oh, s1[...], (((1,), (0,)), ((), ())),
                                 preferred_element_type=jnp.float32)
            rh = lax.dot_general(oh, s2[...], (((1,), (0,)), ((), ())),
                                 preferred_element_type=jnp.float32)
            o_ref[...] = x * rs + rh


def kernel(data, batch_id, batch_size, weights, bias):
    n, c = data.shape
    rows = 2000
    nblocks = n // rows
    assert nblocks * rows == n
    bid = batch_id.astype(jnp.int32)
    bid_col = bid.reshape(n, 1)
    bfirst = bid[::rows]
    blast = bid[rows - 1::rows]

    grid_spec = pltpu.PrefetchScalarGridSpec(
        num_scalar_prefetch=2,
        grid=(2, nblocks),
        in_specs=[
            pl.BlockSpec((rows, c), lambda p, j, bf, bl:
                         (jnp.where(p == 0, j, 0), 0)),
            pl.BlockSpec((rows, 1), lambda p, j, bf, bl: (j, 0)),
            pl.BlockSpec((1, c), lambda p, j, bf, bl: (0, 0)),
            pl.BlockSpec((1, c), lambda p, j, bf, bl: (0, 0)),
        ],
        out_specs=pl.BlockSpec((rows, c), lambda p, j, bf, bl:
                               (jnp.where(p == 0, 0, j), 0)),
        scratch_shapes=[
            pltpu.VMEM((n, c), jnp.float32),
            pltpu.VMEM((NSEG, c), jnp.float32),
            pltpu.VMEM((NSEG, c), jnp.float32),
            pltpu.VMEM((NSEG, c), jnp.float32),
        ],
    )
    out = pl.pallas_call(
        functools.partial(_body, nblocks, rows),
        grid_spec=grid_spec,
        out_shape=jax.ShapeDtypeStruct((n, c), jnp.float32),
        compiler_params=pltpu.CompilerParams(
            dimension_semantics=("arbitrary", "arbitrary")),
    )(bfirst, blast, data, bid_col, weights, bias)
    return out
